# Initial kernel scaffold; baseline (speedup 1.0000x reference)
import jax
import jax.numpy as jnp
from jax.experimental import pallas as pl


def kernel(pred, target, mask_valid):
    raise NotImplementedError("write your pallas kernel here")


# speedup vs baseline: 37.0570x; 37.0570x over previous
import functools

import jax
import jax.numpy as jnp
from jax import lax
from jax.experimental import pallas as pl
from jax.experimental.pallas import tpu as pltpu
from jax.experimental.pallas import tpu_sc as plsc

_NC = 2
_NS = 16
_L = 16
_NW = _NC * _NS
_TOTAL = 32 * 512 * 512
_PER_W = _TOTAL // _NW
_CHUNK = 16384
_NCHUNK = _PER_W // _CHUNK
_VIT = _CHUNK // _L

def _tec_body(pred_hbm, tgt_hbm, msk_hbm, out_hbm,
              p0, p1, t0, t1, m0, m1, outv, sem0, sem1):
    wid = lax.axis_index("s") * _NC + lax.axis_index("c")
    base = wid * _PER_W
    bufs = ((p0, t0, m0), (p1, t1, m1))
    sems = (sem0, sem1)

    def start(g):
        off = base + g * _CHUNK
        b = bufs[g % 2]
        s = sems[g % 2]
        return (pltpu.async_copy(pred_hbm.at[pl.ds(off, _CHUNK)], b[0], s),
                pltpu.async_copy(tgt_hbm.at[pl.ds(off, _CHUNK)], b[1], s),
                pltpu.async_copy(msk_hbm.at[pl.ds(off, _CHUNK)], b[2], s))

    def step_for(bp, bt, bm):
        def step(i, acc):
            a0, a1, a2, a3 = acc
            off = i * _L
            p = bp[pl.ds(off, _L)]
            t = bt[pl.ds(off, _L)]
            m = bm[pl.ds(off, _L)]
            tf = t.astype(jnp.float32)
            mf = m.astype(jnp.float32)
            q = (1.0 - 2.0 * tf) * p
            e = jnp.exp(-jnp.abs(q))
            u = e / (e + 2.0)
            u2 = u * u
            h = jnp.float32(1.0 / 13.0)
            for c in (1.0 / 11.0, 1.0 / 9.0, 1.0 / 7.0,
                      1.0 / 5.0, 1.0 / 3.0, 1.0):
                h = h * u2 + jnp.float32(c)
            sp = jnp.maximum(q, 0.0) + 2.0 * u * h
            v = mf * sp
            return (a0 + mf, a1 + mf * tf, a2 + v, a3 + v * tf)
        return step

    zeros = jnp.zeros((_L,), jnp.float32)
    accs = (zeros, zeros, zeros, zeros)
    cps = start(0)
    for g in range(_NCHUNK):
        nxt = start(g + 1) if g + 1 < _NCHUNK else None
        for cp in cps:
            cp.wait()
        bp, bt, bm = bufs[g % 2]
        accs = lax.fori_loop(0, _VIT, step_for(bp, bt, bm), accs)
        cps = nxt

    a0, a1, a2, a3 = accs
    outv[pl.ds(0, _L)] = a0
    outv[pl.ds(_L, _L)] = a1
    outv[pl.ds(2 * _L, _L)] = a2
    outv[pl.ds(3 * _L, _L)] = a3
    pltpu.sync_copy(outv, out_hbm.at[wid])


@functools.cache
def _sc_partials():
    mesh = plsc.VectorSubcoreMesh(
        core_axis_name="c", subcore_axis_name="s",
        num_cores=_NC, num_subcores=_NS)
    return pl.kernel(
        _tec_body,
        out_type=jax.ShapeDtypeStruct((_NW, 4 * _L), jnp.float32),
        mesh=mesh,
        scratch_types=[
            pltpu.VMEM((_CHUNK,), jnp.float32),
            pltpu.VMEM((_CHUNK,), jnp.float32),
            pltpu.VMEM((_CHUNK,), jnp.int32),
            pltpu.VMEM((_CHUNK,), jnp.int32),
            pltpu.VMEM((_CHUNK,), jnp.int32),
            pltpu.VMEM((_CHUNK,), jnp.int32),
            pltpu.VMEM((4 * _L,), jnp.float32),
            pltpu.SemaphoreType.DMA, pltpu.SemaphoreType.DMA,
        ],
    )


def kernel(pred, target, mask_valid):
    p = pred.reshape(-1)
    t = target.reshape(-1)
    m = mask_valid.reshape(-1)
    parts = _sc_partials()(p, t, m)
    s = parts.reshape(_NW, 4, _L).sum(axis=(0, 2))
    n, c1, S, S1 = s[0], s[1], s[2], s[3]
    counts = jnp.stack([n - c1, c1])
    counts = jnp.where(jnp.isinf(counts), 1.0, counts)
    w = (counts / jnp.sum(counts)) ** (-0.5)
    w = jnp.where(jnp.isinf(w), 1.0, w)
    w = jnp.log(w + 1.1)
    return (w[0] * (S - S1) + w[1] * S1) / n


# speedup vs baseline: 60.1230x; 1.6224x over previous
import functools

import jax
import jax.numpy as jnp
from jax import lax
from jax.experimental import pallas as pl
from jax.experimental.pallas import tpu as pltpu
from jax.experimental.pallas import tpu_sc as plsc

_NC = 2
_NS = 16
_L = 16
_NW = _NC * _NS
_B, _H, _W = 32, 512, 512
_RPC = 32
_NCHUNK = _H // _RPC
_VIT = _RPC * _W // _L
_CPR = _W // _L


def _tec_body(pred_hbm, tgt_hbm, msk_hbm, out_hbm,
              p0, p1, t0, t1, m0, m1, outv, sem0, sem1):
    wid = lax.axis_index("s") * _NC + lax.axis_index("c")
    bufs = ((p0, t0, m0), (p1, t1, m1))
    sems = (sem0, sem1)

    def start(g):
        rows = pl.ds(g * _RPC, _RPC)
        b = bufs[g % 2]
        s = sems[g % 2]
        return (pltpu.async_copy(pred_hbm.at[wid, rows, :], b[0], s),
                pltpu.async_copy(tgt_hbm.at[wid, rows, :], b[1], s),
                pltpu.async_copy(msk_hbm.at[wid, rows, :], b[2], s))

    def step_for(bp, bt, bm):
        def step(i, acc):
            a0, a1, a2, a3 = acc
            r = i >> 5
            c = pl.multiple_of((i & (_CPR - 1)) << 4, _L)
            p = bp[r, pl.ds(c, _L)]
            t = bt[r, pl.ds(c, _L)]
            m = bm[r, pl.ds(c, _L)]
            tf = t.astype(jnp.float32)
            mf = m.astype(jnp.float32)
            q = (1.0 - 2.0 * tf) * p
            e = jnp.exp(-jnp.abs(q))
            u = e / (e + 2.0)
            u2 = u * u
            h = jnp.float32(1.0 / 13.0)
            for cf in (1.0 / 11.0, 1.0 / 9.0, 1.0 / 7.0,
                       1.0 / 5.0, 1.0 / 3.0, 1.0):
                h = h * u2 + jnp.float32(cf)
            sp = jnp.maximum(q, 0.0) + 2.0 * u * h
            v = mf * sp
            return (a0 + mf, a1 + mf * tf, a2 + v, a3 + v * tf)
        return step

    zeros = jnp.zeros((_L,), jnp.float32)
    accs = (zeros, zeros, zeros, zeros)
    cps = start(0)
    for g in range(_NCHUNK):
        nxt = start(g + 1) if g + 1 < _NCHUNK else None
        for cp in cps:
            cp.wait()
        bp, bt, bm = bufs[g % 2]
        accs = lax.fori_loop(0, _VIT, step_for(bp, bt, bm), accs)
        cps = nxt

    a0, a1, a2, a3 = accs
    outv[pl.ds(0, _L)] = a0
    outv[pl.ds(_L, _L)] = a1
    outv[pl.ds(2 * _L, _L)] = a2
    outv[pl.ds(3 * _L, _L)] = a3
    pltpu.sync_copy(outv, out_hbm.at[wid])


@functools.cache
def _sc_partials():
    mesh = plsc.VectorSubcoreMesh(
        core_axis_name="c", subcore_axis_name="s",
        num_cores=_NC, num_subcores=_NS)
    return pl.kernel(
        _tec_body,
        out_type=jax.ShapeDtypeStruct((_NW, 4 * _L), jnp.float32),
        mesh=mesh,
        compiler_params=pltpu.CompilerParams(use_tc_tiling_on_sc=True),
        scratch_types=[
            pltpu.VMEM((_RPC, _W), jnp.float32),
            pltpu.VMEM((_RPC, _W), jnp.float32),
            pltpu.VMEM((_RPC, _W), jnp.int32),
            pltpu.VMEM((_RPC, _W), jnp.int32),
            pltpu.VMEM((_RPC, _W), jnp.int32),
            pltpu.VMEM((_RPC, _W), jnp.int32),
            pltpu.VMEM((4 * _L,), jnp.float32),
            pltpu.SemaphoreType.DMA, pltpu.SemaphoreType.DMA,
        ],
    )


def kernel(pred, target, mask_valid):
    parts = _sc_partials()(pred, target, mask_valid)
    s = parts.reshape(_NW, 4, _L).sum(axis=(0, 2))
    n, c1, S, S1 = s[0], s[1], s[2], s[3]
    counts = jnp.stack([n - c1, c1])
    counts = jnp.where(jnp.isinf(counts), 1.0, counts)
    w = (counts / jnp.sum(counts)) ** (-0.5)
    w = jnp.where(jnp.isinf(w), 1.0, w)
    w = jnp.log(w + 1.1)
    return (w[0] * (S - S1) + w[1] * S1) / n


# speedup vs baseline: 74.2789x; 1.2354x over previous
import functools

import jax
import jax.numpy as jnp
from jax import lax
from jax.experimental import pallas as pl
from jax.experimental.pallas import tpu as pltpu
from jax.experimental.pallas import tpu_sc as plsc

_NC = 2
_NS = 16
_L = 16
_NW = _NC * _NS
_B, _H, _W = 32, 512, 512
_RPC = 32
_NCHUNK = _H // _RPC
_VIT = _RPC * _W // _L
_CPR = _W // _L


def _tec_body(pred_hbm, tgt_hbm, msk_hbm, out_hbm,
              p0, p1, t0, t1, m0, m1, outv, sem0, sem1):
    wid = lax.axis_index("s") * _NC + lax.axis_index("c")
    bufs = ((p0, t0, m0), (p1, t1, m1))
    sems = (sem0, sem1)

    def start(g):
        rows = pl.ds(g * _RPC, _RPC)
        b = bufs[g % 2]
        s = sems[g % 2]
        return (pltpu.async_copy(pred_hbm.at[wid, rows, :], b[0], s),
                pltpu.async_copy(tgt_hbm.at[wid, rows, :], b[1], s),
                pltpu.async_copy(msk_hbm.at[wid, rows, :], b[2], s))

    def step_for(bp, bt, bm):
        def step(i, acc):
            a0, a1, a2, a3 = acc
            r = i >> 5
            c = pl.multiple_of((i & (_CPR - 1)) << 4, _L)
            p = bp[r, pl.ds(c, _L)]
            t = bt[r, pl.ds(c, _L)]
            m = bm[r, pl.ds(c, _L)]
            tm = t != 0
            mm = m != 0
            np_ = -p
            q = jnp.where(tm, np_, p)
            e = jnp.exp(jnp.minimum(p, np_))
            u = e / (e + 2.0)
            u2 = u * u
            L = u * (2.00005181 + u2 * (0.66303484 + u2 * 0.46264232))
            sp = jnp.maximum(q, 0.0) + L
            v = jnp.where(mm, sp, 0.0)
            return (a0 + m, a1 + (m & t),
                    a2 + v, a3 + jnp.where(tm, v, 0.0))
        return step

    zf = jnp.zeros((_L,), jnp.float32)
    zi = jnp.zeros((_L,), jnp.int32)
    accs = (zi, zi, zf, zf)
    cps = start(0)
    for g in range(_NCHUNK):
        nxt = start(g + 1) if g + 1 < _NCHUNK else None
        for cp in cps:
            cp.wait()
        bp, bt, bm = bufs[g % 2]
        accs = lax.fori_loop(0, _VIT, step_for(bp, bt, bm), accs)
        cps = nxt

    a0, a1, a2, a3 = accs
    outv[pl.ds(0, _L)] = a0.astype(jnp.float32)
    outv[pl.ds(_L, _L)] = a1.astype(jnp.float32)
    outv[pl.ds(2 * _L, _L)] = a2
    outv[pl.ds(3 * _L, _L)] = a3
    pltpu.sync_copy(outv, out_hbm.at[wid])


@functools.cache
def _sc_partials():
    mesh = plsc.VectorSubcoreMesh(
        core_axis_name="c", subcore_axis_name="s",
        num_cores=_NC, num_subcores=_NS)
    return pl.kernel(
        _tec_body,
        out_type=jax.ShapeDtypeStruct((_NW, 4 * _L), jnp.float32),
        mesh=mesh,
        compiler_params=pltpu.CompilerParams(use_tc_tiling_on_sc=True),
        scratch_types=[
            pltpu.VMEM((_RPC, _W), jnp.float32),
            pltpu.VMEM((_RPC, _W), jnp.float32),
            pltpu.VMEM((_RPC, _W), jnp.int32),
            pltpu.VMEM((_RPC, _W), jnp.int32),
            pltpu.VMEM((_RPC, _W), jnp.int32),
            pltpu.VMEM((_RPC, _W), jnp.int32),
            pltpu.VMEM((4 * _L,), jnp.float32),
            pltpu.SemaphoreType.DMA, pltpu.SemaphoreType.DMA,
        ],
    )


def kernel(pred, target, mask_valid):
    parts = _sc_partials()(pred, target, mask_valid)
    s = parts.reshape(_NW, 4, _L).sum(axis=(0, 2))
    n, c1, S, S1 = s[0], s[1], s[2], s[3]
    counts = jnp.stack([n - c1, c1])
    counts = jnp.where(jnp.isinf(counts), 1.0, counts)
    w = (counts / jnp.sum(counts)) ** (-0.5)
    w = jnp.where(jnp.isinf(w), 1.0, w)
    w = jnp.log(w + 1.1)
    return (w[0] * (S - S1) + w[1] * S1) / n
